# baseline (speedup 1.0000x reference)
import jax
import jax.numpy as jnp
from jax import lax
from jax.experimental import pallas as pl
from jax.experimental.pallas import tpu as pltpu
from jax.experimental.pallas import tpu_sc as plsc

VOCAB = 1000000
D = 64
BATCH = 16384
SEQ = 50
B = BATCH * SEQ
NC, NS, L = 2, 16, 16
NW = NC * NS
C = 128
NCHUNK = B // (NW * C)
NBUF = 4
ROUNDS = NCHUNK // NBUF


def _body(table_hbm, idx_hbm, scale_hbm, out_hbm,
          idx_all, scale_v, uvs, gbufs, sbufs, gsems, ssems):
    wid = lax.axis_index("s") * NC + lax.axis_index("c")
    cbase = wid * NCHUNK

    pltpu.sync_copy(idx_hbm.at[wid], idx_all)
    pltpu.sync_copy(scale_hbm, scale_v)
    svec = scale_v[...]
    rows = [jnp.arange(16 * g, 16 * (g + 1), dtype=jnp.int32)
            for g in range(C // L)]

    def fire(c, b):
        uv = uvs[b]
        for g in range(C // L):
            sl = pl.ds(g * L, L)
            uv[sl] = lax.shift_right_logical(idx_all[c, sl], 1)
        pltpu.async_copy(table_hbm.at[uv], gbufs[b], gsems[b])

    def wait_gather(b):
        pltpu.make_async_copy(table_hbm.at[uvs[b]], gbufs[b], gsems[b]).wait()

    def select_scale(c, b):
        gbuf, sbuf = gbufs[b], sbufs[b]
        pars = [lax.shift_left(jnp.bitwise_and(idx_all[c, pl.ds(g * L, L)],
                                               1), 6)
                for g in range(C // L)]

        def dstep(d, _):
            for g in range(C // L):
                v = plsc.load_gather(gbuf, [rows[g], pars[g] + d])
                sbuf[d, pl.ds(g * L, L)] = v * svec
            return 0
        lax.fori_loop(0, D, dstep, 0)

    def start_store(c, b):
        g = cbase + c
        s = lax.shift_right_logical(g, 7)
        nb = pl.multiple_of(lax.shift_left(jnp.bitwise_and(g, 127), 7), C)
        pltpu.async_copy(sbufs[b], out_hbm.at[s, :, pl.ds(nb, C)], ssems[b])

    def wait_store(b):
        pltpu.make_async_copy(sbufs[b], out_hbm.at[0, :, pl.ds(0, C)],
                              ssems[b]).wait()

    for b in range(NBUF):
        fire(b, b)

    def step(t, _):
        c0 = t * NBUF
        for b in range(NBUF):
            wait_gather(b)

            @pl.when(t > 0)
            def _():
                wait_store(b)

            select_scale(c0 + b, b)
            start_store(c0 + b, b)

        @pl.when(t < ROUNDS - 1)
        def _():
            for b in range(NBUF):
                fire(c0 + NBUF + b, b)
        return 0

    lax.fori_loop(0, ROUNDS, step, 0)
    for b in range(NBUF):
        wait_store(b)


@jax.jit
def _embed(table2, idx, scale_vec):
    mesh = plsc.VectorSubcoreMesh(core_axis_name="c", subcore_axis_name="s")
    k = pl.kernel(
        _body,
        out_type=jax.ShapeDtypeStruct((SEQ, D, BATCH), jnp.float32),
        mesh=mesh,
        scratch_types=[
            pltpu.VMEM((NCHUNK, C), jnp.int32),
            pltpu.VMEM((L,), jnp.float32),
            [pltpu.VMEM((C,), jnp.int32) for _ in range(NBUF)],
            [pltpu.VMEM((C, 2 * D), jnp.float32) for _ in range(NBUF)],
            [pltpu.VMEM((D, C), jnp.float32) for _ in range(NBUF)],
            [pltpu.SemaphoreType.DMA for _ in range(NBUF)],
            [pltpu.SemaphoreType.DMA for _ in range(NBUF)],
        ],
        compiler_params=pltpu.CompilerParams(use_tc_tiling_on_sc=True,
                                             needs_layout_passes=False),
    )
    return k(table2, idx, scale_vec)


def kernel(token_ids, embed_weight, scale):
    table2 = embed_weight.reshape(VOCAB // 2, 2 * D)
    idx = token_ids.T.reshape(NW, NCHUNK, C).astype(jnp.int32)
    scale_vec = jnp.broadcast_to(scale.astype(jnp.float32), (L,))
    out = _embed(table2, idx, scale_vec)
    return jnp.transpose(out, (2, 0, 1))


# speedup vs baseline: 1.1011x; 1.1011x over previous
import jax
import jax.numpy as jnp
from jax import lax
from jax.experimental import pallas as pl
from jax.experimental.pallas import tpu as pltpu
from jax.experimental.pallas import tpu_sc as plsc

VOCAB = 1000000
D = 64
BATCH = 16384
SEQ = 50
B = BATCH * SEQ
NC, NS, L = 2, 16, 16
NW = NC * NS
C = 64
NCHUNK = B // (NW * C)
NBUF = 4
ROUNDS = NCHUNK // NBUF


def _body(table_hbm, idx_hbm, scale_hbm, out_hbm,
          idx_all, scale_v, uvs, gbufs, sbufs, gsems, ssems):
    wid = lax.axis_index("s") * NC + lax.axis_index("c")
    base = wid * NCHUNK * C

    pltpu.sync_copy(idx_hbm.at[wid], idx_all)
    pltpu.sync_copy(scale_hbm, scale_v)
    svec = scale_v[...]

    def fire(c, b):
        uv = uvs[b]
        for g in range(C // L):
            sl = pl.ds(g * L, L)
            uv[sl] = lax.shift_right_logical(idx_all[c, sl], 1)
        pltpu.async_copy(table_hbm.at[uv], gbufs[b], gsems[b])

    def wait_gather(b):
        pltpu.make_async_copy(table_hbm.at[uvs[b]], gbufs[b], gsems[b]).wait()

    def select_scale(c, b):
        gbuf, sbuf = gbufs[b], sbufs[b]

        def gstep(g, _):
            jb = g * L
            pv = lax.shift_left(
                jnp.bitwise_and(idx_all[c, pl.ds(jb, L)], 1), 6)
            for l in range(L):
                off = pv[l]
                for k in range(D // L):
                    sbuf[jb + l, pl.ds(k * L, L)] = (
                        gbuf[jb + l, pl.ds(off + k * L, L)] * svec)
            return 0
        lax.fori_loop(0, C // L, gstep, 0)

    def start_store(c, b):
        pltpu.async_copy(sbufs[b], out_hbm.at[pl.ds(base + c * C, C)],
                         ssems[b])

    def wait_store(b):
        pltpu.make_async_copy(sbufs[b], out_hbm.at[pl.ds(0, C)],
                              ssems[b]).wait()

    for b in range(NBUF):
        fire(b, b)

    def step(t, _):
        c0 = t * NBUF
        for b in range(NBUF):
            wait_gather(b)

            @pl.when(t > 0)
            def _():
                wait_store(b)

            select_scale(c0 + b, b)
            start_store(c0 + b, b)

        @pl.when(t < ROUNDS - 1)
        def _():
            for b in range(NBUF):
                fire(c0 + NBUF + b, b)
        return 0

    lax.fori_loop(0, ROUNDS, step, 0)
    for b in range(NBUF):
        wait_store(b)


@jax.jit
def _embed(table2, idx, scale_vec):
    mesh = plsc.VectorSubcoreMesh(core_axis_name="c", subcore_axis_name="s")
    k = pl.kernel(
        _body,
        out_type=jax.ShapeDtypeStruct((B, D), jnp.float32),
        mesh=mesh,
        scratch_types=[
            pltpu.VMEM((NCHUNK, C), jnp.int32),
            pltpu.VMEM((L,), jnp.float32),
            [pltpu.VMEM((C,), jnp.int32) for _ in range(NBUF)],
            [pltpu.VMEM((C, 2 * D), jnp.float32) for _ in range(NBUF)],
            [pltpu.VMEM((C, D), jnp.float32) for _ in range(NBUF)],
            [pltpu.SemaphoreType.DMA for _ in range(NBUF)],
            [pltpu.SemaphoreType.DMA for _ in range(NBUF)],
        ],
        compiler_params=pltpu.CompilerParams(use_tc_tiling_on_sc=True,
                                             needs_layout_passes=False),
    )
    return k(table2, idx, scale_vec)


def kernel(token_ids, embed_weight, scale):
    orig_shape = token_ids.shape
    table2 = embed_weight.reshape(VOCAB // 2, 2 * D)
    idx = token_ids.reshape(NW, NCHUNK, C).astype(jnp.int32)
    scale_vec = jnp.broadcast_to(scale.astype(jnp.float32), (L,))
    out = _embed(table2, idx, scale_vec)
    return out.reshape(*orig_shape, D)


# speedup vs baseline: 1.8012x; 1.6359x over previous
import jax
import jax.numpy as jnp
from jax import lax
from jax.experimental import pallas as pl
from jax.experimental.pallas import tpu as pltpu
from jax.experimental.pallas import tpu_sc as plsc

VOCAB = 1000000
D = 64
BATCH = 16384
SEQ = 50
NC, NS, L = 2, 16, 16
NW = NC * NS
NCHUNK = BATCH // NW
NBUF = 4
ROUNDS = NCHUNK // NBUF


def _body(table_hbm, idx_hbm, scale_hbm, out_hbm,
          idx_all, scale_v, gbufs, sbufs, gsems, ssems):
    wid = lax.axis_index("s") * NC + lax.axis_index("c")
    base = wid * NCHUNK

    pltpu.sync_copy(idx_hbm.at[wid], idx_all)
    pltpu.sync_copy(scale_hbm, scale_v)
    svec = scale_v[...]

    def fire(c, b):
        pltpu.async_copy(table_hbm.at[idx_all.at[c]], gbufs[b], gsems[b])

    def wait_gather(b):
        pltpu.make_async_copy(table_hbm.at[idx_all.at[0]],
                              gbufs[b], gsems[b]).wait()

    def select_scale(b):
        gbuf, sbuf = gbufs[b], sbufs[b]

        def jstep(j, _):
            for k in range(D // L):
                sl = pl.ds(k * L, L)
                sbuf[j, sl] = gbuf[j, sl] * svec
            return 0
        lax.fori_loop(0, SEQ, jstep, 0)

    def start_store(c, b):
        pltpu.async_copy(sbufs[b], out_hbm.at[base + c], ssems[b])

    def wait_store(b):
        pltpu.make_async_copy(sbufs[b], out_hbm.at[0], ssems[b]).wait()

    for b in range(NBUF):
        fire(b, b)

    def step(t, _):
        c0 = t * NBUF
        for b in range(NBUF):
            wait_gather(b)

            @pl.when(t > 0)
            def _():
                wait_store(b)

            select_scale(b)
            start_store(c0 + b, b)

        @pl.when(t < ROUNDS - 1)
        def _():
            for b in range(NBUF):
                fire(c0 + NBUF + b, b)
        return 0

    lax.fori_loop(0, ROUNDS, step, 0)
    for b in range(NBUF):
        wait_store(b)


@jax.jit
def _embed(table_pad, idx, scale_vec):
    mesh = plsc.VectorSubcoreMesh(core_axis_name="c", subcore_axis_name="s")
    k = pl.kernel(
        _body,
        out_type=jax.ShapeDtypeStruct((BATCH, SEQ, D), jnp.float32),
        mesh=mesh,
        scratch_types=[
            pltpu.VMEM((NCHUNK, SEQ), jnp.int32),
            pltpu.VMEM((L,), jnp.float32),
            [pltpu.VMEM((SEQ, 2 * D), jnp.float32) for _ in range(NBUF)],
            [pltpu.VMEM((SEQ, D), jnp.float32) for _ in range(NBUF)],
            [pltpu.SemaphoreType.DMA for _ in range(NBUF)],
            [pltpu.SemaphoreType.DMA for _ in range(NBUF)],
        ],
        compiler_params=pltpu.CompilerParams(use_tc_tiling_on_sc=True,
                                             needs_layout_passes=False),
    )
    return k(table_pad, idx, scale_vec)


def kernel(token_ids, embed_weight, scale):
    table_pad = jnp.pad(embed_weight, ((0, 0), (0, D)))
    idx = token_ids.reshape(NW, NCHUNK, SEQ).astype(jnp.int32)
    scale_vec = jnp.broadcast_to(scale.astype(jnp.float32), (L,))
    return _embed(table_pad, idx, scale_vec)
